# baseline (speedup 1.0000x reference)
import jax
import jax.numpy as jnp
from jax.experimental import pallas as pl
from jax.experimental.pallas import tpu as pltpu

_NS = 8


def _gcn_kernel(*refs):
    adj_refs = refs[0:_NS]
    (inputx_ref, wcat_ref, w1_ref, b1_ref, w2_ref,
     nums_ref, b2_ref, wy_ref, by_ref, wa_ref, ba_ref) = refs[_NS:_NS + 11]
    h_out, lsm_out, out_out, aa_out = refs[_NS + 11:_NS + 15]
    xsel_f, c_s, z_s = refs[_NS + 15:]

    p = pl.program_id(0)
    i = pl.program_id(1)
    n_rows = xsel_f.shape[0]
    bm = adj_refs[0].shape[0]
    ck = adj_refs[0].shape[1]

    @pl.when((p == 0) & (i == 0))
    def _build():
        c = jnp.dot(wcat_ref[...], w1_ref[...],
                    preferred_element_type=jnp.float32)
        c_s[...] = c.astype(jnp.bfloat16)
        xin = inputx_ref[...]
        idx = jax.lax.broadcasted_iota(jnp.int32, (n_rows, 1), 0)
        n00 = nums_ref[0, 0]
        n01 = nums_ref[0, 1]
        n10 = nums_ref[1, 0]
        n11 = nums_ref[1, 1]
        n20 = nums_ref[2, 0]
        n21 = nums_ref[2, 1]
        seg2 = n10 != n11
        seg3 = n20 != n21
        mask_r = ((idx < n00)
                  | (seg2 & (idx >= n01) & (idx < n10))
                  | (seg3 & (idx >= n11) & (idx < n20)))
        mask_u = (((idx >= n00) & (idx < n01))
                  | (seg2 & (idx >= n10) & (idx < n11))
                  | (seg3 & (idx >= n20) & (idx < n21)))
        mask_p = idx >= n21
        xsel = jnp.concatenate(
            [xin * mask_r.astype(jnp.float32),
             xin * mask_u.astype(jnp.float32),
             xin * mask_p.astype(jnp.float32)], axis=1)
        xsel_f[...] = xsel

    @pl.when(p == 0)
    def _phase0():
        u = jnp.dot(adj_refs[0][...],
                    xsel_f[pl.ds(0, ck), :],
                    preferred_element_type=jnp.float32)
        for k in range(1, _NS):
            u += jnp.dot(adj_refs[k][...],
                         xsel_f[pl.ds(k * ck, ck), :],
                         preferred_element_type=jnp.float32)
        h = jnp.maximum(
            jnp.dot(u.astype(jnp.bfloat16), c_s[...],
                    preferred_element_type=jnp.float32) + b1_ref[...],
            0.0)
        z = jnp.dot(h.astype(jnp.bfloat16), w2_ref[...],
                    preferred_element_type=jnp.float32)
        z_s[pl.ds(i * bm, bm), :] = z

    @pl.when(p == 1)
    def _phase1():
        h2 = jnp.dot(adj_refs[0][...],
                     z_s[pl.ds(0, ck), :],
                     preferred_element_type=jnp.float32)
        for k in range(1, _NS):
            h2 += jnp.dot(adj_refs[k][...],
                          z_s[pl.ds(k * ck, ck), :],
                          preferred_element_type=jnp.float32)
        c0 = h2[:, 0:1] + b2_ref[0, 0]
        c1 = h2[:, 1:2] + b2_ref[0, 1]
        h_out[:, 0:1] = c0
        h_out[:, 1:2] = c1
        m = jnp.maximum(c0, c1)
        lse = m + jnp.log(jnp.exp(c0 - m) + jnp.exp(c1 - m))
        lsm_out[:, 0:1] = c0 - lse
        lsm_out[:, 1:2] = c1 - lse
        y0 = c0 * wy_ref[0, 0] + c1 * wy_ref[1, 0] + by_ref[0, 0]
        y1 = c0 * wy_ref[0, 1] + c1 * wy_ref[1, 1] + by_ref[0, 1]
        out_out[:, 0:1] = jax.nn.sigmoid(y0)
        out_out[:, 1:2] = jax.nn.sigmoid(y1)
        a0 = c0 * wa_ref[0, 0] + c1 * wa_ref[1, 0] + ba_ref[0, 0]
        a1 = c0 * wa_ref[0, 1] + c1 * wa_ref[1, 1] + ba_ref[0, 1]
        aa_out[:, 0:1] = jax.nn.sigmoid(a0)
        aa_out[:, 1:2] = jax.nn.sigmoid(a1)


def kernel(inputx, adj, nums, Wr, br, Wu, bu, Wp, bp, W1, b1, W2, b2,
           Wy, by, Wa, ba):
    n = adj.shape[0]
    f = W1.shape[0]
    bm = 512
    nblk = n // bm
    ck = n // _NS

    xin = jnp.concatenate(
        [inputx, jnp.ones((n, 1), dtype=inputx.dtype)], axis=1)
    zrow = jnp.zeros((f,), dtype=W1.dtype)
    wcat = jnp.stack([
        Wr[0], Wr[1], Wr[2], Wr[3], Wr[4], zrow, zrow, br,
        Wu[0], Wu[1], Wu[2], Wu[3], Wu[4], Wu[5], Wu[6], bu,
        Wp[0], Wp[1], Wp[2], Wp[3], Wp[4], Wp[5], zrow, bp,
    ])
    w2b = W2.astype(jnp.bfloat16)

    vspec_whole = lambda shape: pl.BlockSpec(
        shape, lambda p, i: tuple(0 for _ in shape))
    smem = pl.BlockSpec(memory_space=pltpu.SMEM)
    out_spec = pl.BlockSpec((bm, 2), lambda p, i: (p * i, 0))
    adj_specs = [
        pl.BlockSpec((bm, ck), lambda p, i, k=k: (i, k)) for k in range(_NS)
    ]

    outs = pl.pallas_call(
        _gcn_kernel,
        grid=(2, nblk),
        in_specs=adj_specs + [
            vspec_whole((n, 8)),
            vspec_whole((24, f)),
            vspec_whole((f, f)),
            vspec_whole((1, f)),
            vspec_whole((f, 2)),
            smem,
            smem,
            smem,
            smem,
            smem,
            smem,
        ],
        out_specs=[out_spec, out_spec, out_spec, out_spec],
        out_shape=[jax.ShapeDtypeStruct((n, 2), jnp.float32)] * 4,
        scratch_shapes=[
            pltpu.VMEM((n, 24), jnp.float32),
            pltpu.VMEM((24, f), jnp.bfloat16),
            pltpu.VMEM((n, 2), jnp.float32),
        ],
        compiler_params=pltpu.CompilerParams(
            dimension_semantics=("arbitrary", "arbitrary")),
    )(*([adj] * _NS), xin, wcat, W1, b1.reshape(1, f), w2b,
      nums, b2.reshape(1, 2), Wy, by.reshape(1, 2), Wa, ba.reshape(1, 2))
    h, lsm, out, aa = outs
    return (h, lsm, out, aa)


# speedup vs baseline: 1.0687x; 1.0687x over previous
import jax
import jax.numpy as jnp
from jax.experimental import pallas as pl
from jax.experimental.pallas import tpu as pltpu


def _gcn_kernel(adj_ref, inputx_ref, wr_ref, wu_ref, wp_ref, w1_ref,
                br_ref, bu_ref, bp_ref, b1_ref, w2_ref,
                nums_ref, b2_ref, wy_ref, by_ref, wa_ref, ba_ref,
                h_out, lsm_out, out_out, aa_out,
                xsel_s, wcat_s, c_s, z_s):
    p = pl.program_id(0)
    i = pl.program_id(1)
    n_rows = xsel_s.shape[0]
    bm = adj_ref.shape[0]

    @pl.when((p == 0) & (i == 0))
    def _build():
        wcat_s[...] = jnp.zeros(wcat_s.shape, jnp.float32)
        wcat_s[0:5, :] = wr_ref[...]
        wcat_s[7:8, :] = br_ref[...]
        wcat_s[8:15, :] = wu_ref[...]
        wcat_s[15:16, :] = bu_ref[...]
        wcat_s[16:22, :] = wp_ref[...]
        wcat_s[23:24, :] = bp_ref[...]
        c_s[...] = jnp.dot(wcat_s[...], w1_ref[...],
                           preferred_element_type=jnp.float32)
        xin = jnp.concatenate(
            [inputx_ref[...], jnp.ones((n_rows, 1), jnp.float32)], axis=1)
        idx = jax.lax.broadcasted_iota(jnp.int32, (n_rows, 1), 0)
        n00 = nums_ref[0, 0]
        n01 = nums_ref[0, 1]
        n10 = nums_ref[1, 0]
        n11 = nums_ref[1, 1]
        n20 = nums_ref[2, 0]
        n21 = nums_ref[2, 1]
        seg2 = n10 != n11
        seg3 = n20 != n21
        mask_r = ((idx < n00)
                  | (seg2 & (idx >= n01) & (idx < n10))
                  | (seg3 & (idx >= n11) & (idx < n20)))
        mask_u = (((idx >= n00) & (idx < n01))
                  | (seg2 & (idx >= n10) & (idx < n11))
                  | (seg3 & (idx >= n20) & (idx < n21)))
        mask_p = idx >= n21
        xsel_s[...] = jnp.concatenate(
            [xin * mask_r.astype(jnp.float32),
             xin * mask_u.astype(jnp.float32),
             xin * mask_p.astype(jnp.float32)], axis=1)

    @pl.when(p == 0)
    def _phase0():
        u = jnp.dot(adj_ref[...], xsel_s[...],
                    preferred_element_type=jnp.float32)
        h = jnp.maximum(
            jnp.dot(u, c_s[...], preferred_element_type=jnp.float32)
            + b1_ref[...],
            0.0)
        z = jnp.dot(h, w2_ref[...], preferred_element_type=jnp.float32)
        z_s[pl.ds(i * bm, bm), :] = z

    @pl.when(p == 1)
    def _phase1():
        h2 = jnp.dot(adj_ref[...], z_s[...],
                     preferred_element_type=jnp.float32)
        c0 = h2[:, 0:1] + b2_ref[0, 0]
        c1 = h2[:, 1:2] + b2_ref[0, 1]
        h_out[:, 0:1] = c0
        h_out[:, 1:2] = c1
        m = jnp.maximum(c0, c1)
        lse = m + jnp.log(jnp.exp(c0 - m) + jnp.exp(c1 - m))
        lsm_out[:, 0:1] = c0 - lse
        lsm_out[:, 1:2] = c1 - lse
        y0 = c0 * wy_ref[0, 0] + c1 * wy_ref[1, 0] + by_ref[0, 0]
        y1 = c0 * wy_ref[0, 1] + c1 * wy_ref[1, 1] + by_ref[0, 1]
        out_out[:, 0:1] = jax.nn.sigmoid(y0)
        out_out[:, 1:2] = jax.nn.sigmoid(y1)
        a0 = c0 * wa_ref[0, 0] + c1 * wa_ref[1, 0] + ba_ref[0, 0]
        a1 = c0 * wa_ref[0, 1] + c1 * wa_ref[1, 1] + ba_ref[0, 1]
        aa_out[:, 0:1] = jax.nn.sigmoid(a0)
        aa_out[:, 1:2] = jax.nn.sigmoid(a1)


def kernel(inputx, adj, nums, Wr, br, Wu, bu, Wp, bp, W1, b1, W2, b2,
           Wy, by, Wa, ba):
    n = adj.shape[0]
    f = W1.shape[0]
    bm = 512
    nblk = n // bm

    vspec_whole = lambda shape: pl.BlockSpec(
        shape, lambda p, i: tuple(0 for _ in shape))
    smem = pl.BlockSpec(memory_space=pltpu.SMEM)
    out_spec = pl.BlockSpec((bm, 2), lambda p, i: (p * i, 0))

    outs = pl.pallas_call(
        _gcn_kernel,
        grid=(2, nblk),
        in_specs=[
            pl.BlockSpec((bm, n), lambda p, i: (i, 0)),
            vspec_whole((n, 7)),
            vspec_whole((5, f)),
            vspec_whole((7, f)),
            vspec_whole((6, f)),
            vspec_whole((f, f)),
            vspec_whole((1, f)),
            vspec_whole((1, f)),
            vspec_whole((1, f)),
            vspec_whole((1, f)),
            vspec_whole((f, 2)),
            smem,
            smem,
            smem,
            smem,
            smem,
            smem,
        ],
        out_specs=[out_spec, out_spec, out_spec, out_spec],
        out_shape=[jax.ShapeDtypeStruct((n, 2), jnp.float32)] * 4,
        scratch_shapes=[
            pltpu.VMEM((n, 24), jnp.float32),
            pltpu.VMEM((24, f), jnp.float32),
            pltpu.VMEM((24, f), jnp.float32),
            pltpu.VMEM((n, 2), jnp.float32),
        ],
        compiler_params=pltpu.CompilerParams(
            dimension_semantics=("arbitrary", "arbitrary")),
    )(adj, inputx, Wr, Wu, Wp, W1,
      br.reshape(1, f), bu.reshape(1, f), bp.reshape(1, f), b1.reshape(1, f),
      W2, nums, b2.reshape(1, 2), Wy, by.reshape(1, 2), Wa, ba.reshape(1, 2))
    h, lsm, out, aa = outs
    return (h, lsm, out, aa)
